# baseline (speedup 1.0000x reference)
import functools

import jax
import jax.numpy as jnp
from jax import lax
from jax.experimental import pallas as pl
from jax.experimental.pallas import tpu as pltpu
from jax.experimental.pallas import tpu_sc as plsc

DIM = 128
SAMP = 4096
SEQ = 50
SEQP = 56
NC = 2
NS = 16
NW = NC * NS
SPW = SAMP // NW
S_CH = 4
NRING = 4
NCH = SPW // S_CH
NROUND = NCH // NRING

_mesh = plsc.VectorSubcoreMesh(core_axis_name="c", subcore_axis_name="s")


@functools.partial(
    pl.kernel,
    mesh=_mesh,
    out_type=jax.ShapeDtypeStruct((SAMP, SEQ, DIM), jnp.float32),
    scratch_types=(
        [pltpu.VMEM((S_CH, SEQ), jnp.int32) for _ in range(NRING)]
        + [pltpu.VMEM((S_CH, SEQP, DIM), jnp.float32) for _ in range(NRING)]
        + [pltpu.SemaphoreType.DMA for _ in range(2 * NRING)]
    ),
)
def _gather_kernel(idx_hbm, table_hbm, out_hbm, *refs):
    idx_r = refs[0:NRING]
    rows_r = refs[NRING:2 * NRING]
    gsem_r = refs[2 * NRING:3 * NRING]
    ssem_r = refs[3 * NRING:4 * NRING]
    wid = lax.axis_index("s") * NC + lax.axis_index("c")
    base_s = wid * SPW

    def stage(s0, p):
        pltpu.sync_copy(idx_hbm.at[pl.ds(s0, S_CH)], idx_r[p])
        for k in range(S_CH):
            pltpu.async_copy(
                table_hbm.at[idx_r[p].at[k]],
                rows_r[p].at[k, pl.ds(0, SEQ)],
                gsem_r[p],
            )

    def drain_gathers(p):
        pltpu.make_async_copy(
            out_hbm.at[pl.ds(0, S_CH)],
            rows_r[p].at[pl.ds(0, S_CH), pl.ds(0, SEQ)],
            gsem_r[p],
        ).wait()

    for p in range(NRING):
        stage(base_s + p * S_CH, p)

    def round_(g, carry):
        writes = []
        for p in range(NRING):
            drain_gathers(p)
            s0 = base_s + (g * NRING + p) * S_CH
            writes.append(
                pltpu.async_copy(
                    rows_r[p].at[pl.ds(0, S_CH), pl.ds(0, SEQ)],
                    out_hbm.at[pl.ds(s0, S_CH)],
                    ssem_r[p],
                )
            )
        for p in range(NRING):
            writes[p].wait()
            nxt = jnp.minimum(g * NRING + NRING + p, NCH - 1)
            stage(base_s + nxt * S_CH, p)
        return carry

    lax.fori_loop(0, NROUND, round_, 0)

    for p in range(NRING):
        drain_gathers(p)


def kernel(input, emb_weight):
    return _gather_kernel(input.astype(jnp.int32), emb_weight)


# speedup vs baseline: 1.0174x; 1.0174x over previous
import functools

import jax
import jax.numpy as jnp
from jax import lax
from jax.experimental import pallas as pl
from jax.experimental.pallas import tpu as pltpu
from jax.experimental.pallas import tpu_sc as plsc

DIM = 128
SAMP = 4096
SEQ = 50
SEQP = 56
NC = 2
NS = 16
NW = NC * NS
SPW = SAMP // NW
S_CH = 8
NPAIR = SPW // (2 * S_CH)

_mesh = plsc.VectorSubcoreMesh(core_axis_name="c", subcore_axis_name="s")


@functools.partial(
    pl.kernel,
    mesh=_mesh,
    out_type=jax.ShapeDtypeStruct((SAMP, SEQ, DIM), jnp.float32),
    scratch_types=[
        pltpu.VMEM((S_CH, SEQ), jnp.int32),
        pltpu.VMEM((S_CH, SEQ), jnp.int32),
        pltpu.VMEM((S_CH, SEQP, DIM), jnp.float32),
        pltpu.VMEM((S_CH, SEQP, DIM), jnp.float32),
        pltpu.SemaphoreType.DMA,
        pltpu.SemaphoreType.DMA,
        pltpu.SemaphoreType.DMA,
        pltpu.SemaphoreType.DMA,
    ],
)
def _gather_kernel(idx_hbm, table_hbm, out_hbm, idx_a, idx_b, rows_a,
                   rows_b, gsem_a, gsem_b, ssem_a, ssem_b):
    wid = lax.axis_index("s") * NC + lax.axis_index("c")
    base_s = wid * SPW

    def stage(s0, idx_v, rows_v, gsem):
        pltpu.sync_copy(idx_hbm.at[pl.ds(s0, S_CH)], idx_v)
        return [
            pltpu.async_copy(
                table_hbm.at[idx_v.at[k]],
                rows_v.at[k, pl.ds(0, SEQ)],
                gsem,
            )
            for k in range(S_CH)
        ]

    def writeback(s0, rows_v, ssem):
        return pltpu.async_copy(
            rows_v.at[pl.ds(0, S_CH), pl.ds(0, SEQ)],
            out_hbm.at[pl.ds(s0, S_CH)],
            ssem,
        )

    def pair(g, carry):
        sa = base_s + g * (2 * S_CH)
        sb = sa + S_CH
        ga = stage(sa, idx_a, rows_a, gsem_a)
        gb = stage(sb, idx_b, rows_b, gsem_b)
        for cp in ga:
            cp.wait()
        wa = writeback(sa, rows_a, ssem_a)
        for cp in gb:
            cp.wait()
        wb = writeback(sb, rows_b, ssem_b)
        wa.wait()
        wb.wait()
        return carry

    lax.fori_loop(0, NPAIR, pair, 0)


def kernel(input, emb_weight):
    return _gather_kernel(input.astype(jnp.int32), emb_weight)
